# baseline (speedup 1.0000x reference)
import functools

import jax
import jax.numpy as jnp
from jax import lax
from jax.experimental import pallas as pl
from jax.experimental.pallas import tpu as pltpu
from jax.experimental.pallas import tpu_sc as plsc

N = 10000
E = 320000
NC = 2
NS = 16
NW = NC * NS
EDGES_PER_TILE = E // NW
CHUNK = 96
NCHUNK = 105
EPT_PAD = NCHUNK * CHUNK
NPAD = 10240
ROWS_PER_TILE = NPAD // NS
ZROWS = 32
LANES = 16


def _mm_body(x_ref, w_ref, o_ref):
    o_ref[...] = jnp.dot(x_ref[...], w_ref[...],
                         preferred_element_type=jnp.float32)


def _matmul(x, w, blk=1000):
    n, k = x.shape
    m = w.shape[1]
    return pl.pallas_call(
        _mm_body,
        grid=(n // blk,),
        in_specs=[pl.BlockSpec((blk, k), lambda i: (i, 0)),
                  pl.BlockSpec((k, m), lambda i: (0, 0))],
        out_specs=pl.BlockSpec((blk, m), lambda i: (i, 0)),
        out_shape=jax.ShapeDtypeStruct((n, m), jnp.float32),
    )(x, w)


def _relu_add_body(p_ref, o_ref):
    o_ref[...] = jnp.maximum(p_ref[0] + p_ref[1], 0.0)


def _relu_add(p, blk=1000):
    _, _, k = p.shape
    n = N
    return pl.pallas_call(
        _relu_add_body,
        grid=(n // blk,),
        in_specs=[pl.BlockSpec((2, blk, k), lambda i: (0, i, 0))],
        out_specs=pl.BlockSpec((blk, k), lambda i: (i, 0)),
        out_shape=jax.ShapeDtypeStruct((n, k), jnp.float32),
    )(p)


def _add_mm_body(q_ref, w_ref, o_ref):
    o_ref[...] = jnp.dot(q_ref[0] + q_ref[1], w_ref[...],
                         preferred_element_type=jnp.float32)


def _add_mm(q, w, blk=1000):
    _, _, k = q.shape
    n = N
    m = w.shape[1]
    return pl.pallas_call(
        _add_mm_body,
        grid=(n // blk,),
        in_specs=[pl.BlockSpec((2, blk, k), lambda i: (0, i, 0)),
                  pl.BlockSpec((k, m), lambda i: (0, 0))],
        out_specs=pl.BlockSpec((blk, m), lambda i: (i, 0)),
        out_shape=jax.ShapeDtypeStruct((n, m), jnp.float32),
    )(q, w)


def _make_agg(F):
    mesh = plsc.VectorSubcoreMesh(core_axis_name="c", subcore_axis_name="s")
    f_vecs = F // LANES

    @functools.partial(
        pl.kernel,
        out_type=jax.ShapeDtypeStruct((NC, NPAD, F), jnp.float32),
        mesh=mesh,
        scratch_types=[
            pltpu.VMEM((CHUNK,), jnp.int32),
            pltpu.VMEM((CHUNK,), jnp.int32),
            pltpu.VMEM((CHUNK,), jnp.int32),
            pltpu.VMEM((CHUNK,), jnp.int32),
            pltpu.VMEM((CHUNK, F), jnp.float32),
            pltpu.VMEM((CHUNK, F), jnp.float32),
            pltpu.VMEM((ZROWS, F), jnp.float32),
            pltpu.VMEM_SHARED((NPAD, F), jnp.float32),
            pltpu.SemaphoreType.DMA,
            pltpu.SemaphoreType.DMA,
            pltpu.SemaphoreType.DMA,
            pltpu.SemaphoreType.DMA,
            pltpu.SemaphoreType.DMA,
            pltpu.SemaphoreType.DMA,
        ],
    )
    def agg(table_hbm, src_hbm, dst_hbm, out_hbm,
            src_a, dst_a, src_b, dst_b, rows_a, rows_b, zbuf, acc,
            sg_a, sg_b, ss_a, ss_b, si_a, si_b):
        cid = lax.axis_index("c")
        sid = lax.axis_index("s")
        wid = cid * NS + sid

        zero = jnp.zeros((LANES,), jnp.float32)

        def zbody(t, carry):
            zbuf[t // f_vecs, pl.ds((t % f_vecs) * LANES, LANES)] = zero
            return carry

        lax.fori_loop(0, ZROWS * f_vecs, zbody, 0)

        def zcopy(i, carry):
            pltpu.sync_copy(
                zbuf, acc.at[pl.ds(sid * ROWS_PER_TILE + i * ZROWS, ZROWS)])
            return carry

        lax.fori_loop(0, ROWS_PER_TILE // ZROWS, zcopy, 0)
        plsc.subcore_barrier()

        tile_base = wid * EPT_PAD
        last = NCHUNK - 1

        def sbase(k):
            return tile_base + jnp.minimum(k, last) * CHUNK

        pltpu.async_copy(src_hbm.at[pl.ds(sbase(0), CHUNK)], src_a, si_a)
        pltpu.async_copy(dst_hbm.at[pl.ds(sbase(0), CHUNK)], dst_a, si_a)
        pltpu.async_copy(src_hbm.at[pl.ds(sbase(1), CHUNK)], src_b, si_b)
        pltpu.make_async_copy(src_hbm.at[pl.ds(0, CHUNK)], src_a, si_a).wait()
        pltpu.make_async_copy(dst_hbm.at[pl.ds(0, CHUNK)], dst_a, si_a).wait()
        pltpu.async_copy(table_hbm.at[src_a], rows_a, sg_a)

        def pair(j, carry):
            k = 2 * j

            @pl.when(j > 0)
            def _():
                pltpu.make_async_copy(rows_b, acc.at[dst_b], ss_b).wait()
            pltpu.async_copy(dst_hbm.at[pl.ds(sbase(k + 1), CHUNK)], dst_b, si_b)
            pltpu.make_async_copy(table_hbm.at[src_a], rows_a, sg_a).wait()
            pltpu.async_copy(src_hbm.at[pl.ds(sbase(k + 2), CHUNK)], src_a, si_a)
            pltpu.async_copy(rows_a, acc.at[dst_a], ss_a, add=True)
            pltpu.make_async_copy(src_hbm.at[pl.ds(0, CHUNK)], src_b, si_b).wait()
            pltpu.make_async_copy(dst_hbm.at[pl.ds(0, CHUNK)], dst_b, si_b).wait()
            pltpu.async_copy(table_hbm.at[src_b], rows_b, sg_b)
            pltpu.make_async_copy(rows_a, acc.at[dst_a], ss_a).wait()
            pltpu.async_copy(dst_hbm.at[pl.ds(sbase(k + 2), CHUNK)], dst_a, si_a)
            pltpu.make_async_copy(table_hbm.at[src_b], rows_b, sg_b).wait()
            pltpu.async_copy(src_hbm.at[pl.ds(sbase(k + 3), CHUNK)], src_b, si_b)
            pltpu.async_copy(rows_b, acc.at[dst_b], ss_b, add=True)
            pltpu.make_async_copy(src_hbm.at[pl.ds(0, CHUNK)], src_a, si_a).wait()
            pltpu.make_async_copy(dst_hbm.at[pl.ds(0, CHUNK)], dst_a, si_a).wait()
            pltpu.async_copy(table_hbm.at[src_a], rows_a, sg_a)
            return carry

        lax.fori_loop(0, NCHUNK // 2, pair, 0)
        pltpu.make_async_copy(rows_b, acc.at[dst_b], ss_b).wait()
        pltpu.make_async_copy(src_hbm.at[pl.ds(0, CHUNK)], src_b, si_b).wait()
        pltpu.make_async_copy(table_hbm.at[src_a], rows_a, sg_a).wait()
        pltpu.sync_copy(rows_a, acc.at[dst_a], add=True)
        plsc.subcore_barrier()

        pltpu.sync_copy(acc.at[pl.ds(sid * ROWS_PER_TILE, ROWS_PER_TILE)],
                        out_hbm.at[cid, pl.ds(sid * ROWS_PER_TILE, ROWS_PER_TILE)])

    return agg


_agg128 = _make_agg(128)


def _pad_edges(v, fill):
    t = v.reshape(NW, EDGES_PER_TILE)
    pad = jnp.full((NW, EPT_PAD - EDGES_PER_TILE), fill, v.dtype)
    return jnp.concatenate([t, pad], axis=1).reshape(-1)


def kernel(x, edge_index, W1, W2):
    dst = _pad_edges(edge_index[0], N)
    src = _pad_edges(edge_index[1], 0)
    support1 = _matmul(x, W1)
    p = _agg128(support1, src, dst)
    h1 = _relu_add(p)
    q = _agg128(h1, src, dst)
    return _add_mm(q, W2)


# speedup vs baseline: 1.4268x; 1.4268x over previous
import functools

import jax
import jax.numpy as jnp
from jax import lax
from jax.experimental import pallas as pl
from jax.experimental.pallas import tpu as pltpu
from jax.experimental.pallas import tpu_sc as plsc

N = 10000
E = 320000
NC = 2
NS = 16
NW = NC * NS
EDGES_PER_TILE = E // NW
CHUNK = 80
NCHUNK = 125
EPT_PAD = NCHUNK * CHUNK
NPAD = 10240
ROWS_PER_TILE = NPAD // NS
ZROWS = 32
LANES = 16


def _mm_body(x_ref, w_ref, o_ref):
    o_ref[...] = jnp.dot(x_ref[...], w_ref[...],
                         preferred_element_type=jnp.float32)


def _matmul(x, w, blk=1000):
    n, k = x.shape
    m = w.shape[1]
    return pl.pallas_call(
        _mm_body,
        grid=(n // blk,),
        in_specs=[pl.BlockSpec((blk, k), lambda i: (i, 0)),
                  pl.BlockSpec((k, m), lambda i: (0, 0))],
        out_specs=pl.BlockSpec((blk, m), lambda i: (i, 0)),
        out_shape=jax.ShapeDtypeStruct((n, m), jnp.float32),
    )(x, w)


def _relu_add_body(p_ref, o_ref):
    o_ref[...] = jnp.maximum(p_ref[0] + p_ref[1], 0.0)


def _relu_add(p, blk=1000):
    _, _, k = p.shape
    n = N
    return pl.pallas_call(
        _relu_add_body,
        grid=(n // blk,),
        in_specs=[pl.BlockSpec((2, blk, k), lambda i: (0, i, 0))],
        out_specs=pl.BlockSpec((blk, k), lambda i: (i, 0)),
        out_shape=jax.ShapeDtypeStruct((n, k), jnp.float32),
    )(p)


def _add_mm_body(q_ref, w_ref, o_ref):
    o_ref[...] = jnp.dot(q_ref[0] + q_ref[1], w_ref[...],
                         preferred_element_type=jnp.float32)


def _add_mm(q, w, blk=1000):
    _, _, k = q.shape
    n = N
    m = w.shape[1]
    return pl.pallas_call(
        _add_mm_body,
        grid=(n // blk,),
        in_specs=[pl.BlockSpec((2, blk, k), lambda i: (0, i, 0)),
                  pl.BlockSpec((k, m), lambda i: (0, 0))],
        out_specs=pl.BlockSpec((blk, m), lambda i: (i, 0)),
        out_shape=jax.ShapeDtypeStruct((n, m), jnp.float32),
    )(q, w)


def _make_agg(F):
    mesh = plsc.VectorSubcoreMesh(core_axis_name="c", subcore_axis_name="s")
    f_vecs = F // LANES

    @functools.partial(
        pl.kernel,
        out_type=jax.ShapeDtypeStruct((NC, NPAD, F), jnp.float32),
        mesh=mesh,
        scratch_types=[
            pltpu.VMEM((CHUNK,), jnp.int32),
            pltpu.VMEM((CHUNK,), jnp.int32),
            pltpu.VMEM((CHUNK,), jnp.int32),
            pltpu.VMEM((CHUNK,), jnp.int32),
            pltpu.VMEM((CHUNK, F), jnp.float32),
            pltpu.VMEM((CHUNK, F), jnp.float32),
            pltpu.VMEM((ZROWS, F), jnp.float32),
            pltpu.VMEM_SHARED((NPAD, F), jnp.float32),
            pltpu.SemaphoreType.DMA,
            pltpu.SemaphoreType.DMA,
            pltpu.SemaphoreType.DMA,
            pltpu.SemaphoreType.DMA,
            pltpu.SemaphoreType.DMA,
            pltpu.SemaphoreType.DMA,
        ],
    )
    def agg(table_hbm, src_hbm, dst_hbm, out_hbm,
            src_a, dst_a, src_b, dst_b, rows_a, rows_b, zbuf, acc,
            sg_a, sg_b, ss_a, ss_b, si_a, si_b):
        cid = lax.axis_index("c")
        sid = lax.axis_index("s")
        wid = cid * NS + sid

        zero = jnp.zeros((LANES,), jnp.float32)

        def zbody(t, carry):
            zbuf[t // f_vecs, pl.ds((t % f_vecs) * LANES, LANES)] = zero
            return carry

        lax.fori_loop(0, ZROWS * f_vecs, zbody, 0)

        def zcopy(i, carry):
            pltpu.async_copy(
                zbuf, acc.at[pl.ds(sid * ROWS_PER_TILE + i * ZROWS, ZROWS)],
                si_a)
            return carry

        lax.fori_loop(0, ROWS_PER_TILE // ZROWS, zcopy, 0)

        def zwait(i, carry):
            pltpu.make_async_copy(
                zbuf, acc.at[pl.ds(sid * ROWS_PER_TILE, ZROWS)], si_a).wait()
            return carry

        lax.fori_loop(0, ROWS_PER_TILE // ZROWS, zwait, 0)
        plsc.subcore_barrier()

        tile_base = wid * EPT_PAD
        last = NCHUNK - 1

        def sbase(k):
            return tile_base + jnp.minimum(k, last) * CHUNK

        pltpu.async_copy(src_hbm.at[pl.ds(sbase(0), CHUNK)], src_a, si_a)
        pltpu.async_copy(dst_hbm.at[pl.ds(sbase(0), CHUNK)], dst_a, si_a)
        pltpu.async_copy(src_hbm.at[pl.ds(sbase(1), CHUNK)], src_b, si_b)
        pltpu.make_async_copy(src_hbm.at[pl.ds(0, CHUNK)], src_a, si_a).wait()
        pltpu.make_async_copy(dst_hbm.at[pl.ds(0, CHUNK)], dst_a, si_a).wait()
        pltpu.async_copy(table_hbm.at[src_a], rows_a, sg_a)

        def pair(j, carry):
            k = 2 * j

            @pl.when(j > 0)
            def _():
                pltpu.make_async_copy(rows_b, acc.at[dst_b], ss_b).wait()
            pltpu.async_copy(dst_hbm.at[pl.ds(sbase(k + 1), CHUNK)], dst_b, si_b)
            pltpu.make_async_copy(table_hbm.at[src_a], rows_a, sg_a).wait()
            pltpu.async_copy(src_hbm.at[pl.ds(sbase(k + 2), CHUNK)], src_a, si_a)
            pltpu.async_copy(rows_a, acc.at[dst_a], ss_a, add=True)
            pltpu.make_async_copy(src_hbm.at[pl.ds(0, CHUNK)], src_b, si_b).wait()
            pltpu.make_async_copy(dst_hbm.at[pl.ds(0, CHUNK)], dst_b, si_b).wait()
            pltpu.async_copy(table_hbm.at[src_b], rows_b, sg_b)
            pltpu.make_async_copy(rows_a, acc.at[dst_a], ss_a).wait()
            pltpu.async_copy(dst_hbm.at[pl.ds(sbase(k + 2), CHUNK)], dst_a, si_a)
            pltpu.make_async_copy(table_hbm.at[src_b], rows_b, sg_b).wait()
            pltpu.async_copy(src_hbm.at[pl.ds(sbase(k + 3), CHUNK)], src_b, si_b)
            pltpu.async_copy(rows_b, acc.at[dst_b], ss_b, add=True)
            pltpu.make_async_copy(src_hbm.at[pl.ds(0, CHUNK)], src_a, si_a).wait()
            pltpu.make_async_copy(dst_hbm.at[pl.ds(0, CHUNK)], dst_a, si_a).wait()
            pltpu.async_copy(table_hbm.at[src_a], rows_a, sg_a)
            return carry

        lax.fori_loop(0, NCHUNK // 2, pair, 0)
        pltpu.make_async_copy(rows_b, acc.at[dst_b], ss_b).wait()
        pltpu.make_async_copy(src_hbm.at[pl.ds(0, CHUNK)], src_b, si_b).wait()
        pltpu.make_async_copy(table_hbm.at[src_a], rows_a, sg_a).wait()
        pltpu.sync_copy(rows_a, acc.at[dst_a], add=True)
        plsc.subcore_barrier()

        pltpu.sync_copy(acc.at[pl.ds(sid * ROWS_PER_TILE, ROWS_PER_TILE)],
                        out_hbm.at[cid, pl.ds(sid * ROWS_PER_TILE, ROWS_PER_TILE)])

    return agg


_agg128 = _make_agg(128)


def _pad_edges(v, fill):
    t = v.reshape(NW, EDGES_PER_TILE)
    pad = jnp.full((NW, EPT_PAD - EDGES_PER_TILE), fill, v.dtype)
    return jnp.concatenate([t, pad], axis=1).reshape(-1)


def kernel(x, edge_index, W1, W2):
    dst = _pad_edges(edge_index[0], N)
    src = _pad_edges(edge_index[1], 0)
    support1 = _matmul(x, W1)
    p = _agg128(support1, src, dst)
    h1 = _relu_add(p)
    q = _agg128(h1, src, dst)
    return _add_mm(q, W2)
